# baseline (speedup 1.0000x reference)
import jax
import jax.numpy as jnp
from jax import lax
from jax.experimental import pallas as pl
from jax.experimental.pallas import tpu as pltpu
from jax.experimental.pallas import tpu_sc as plsc

_LAMB_COORD = 5.0
_LAMB_NOOBJ = 0.5
_B, _C, _S = 2048, 30, 7
_S2 = _S * _S
_BW = 512
_NBP = _B // _BW
_NTASK = _S2 * _NBP
_NW = 32
_SH = 2
_PAIRS = 3


def _sqrt16(x):
    x = jnp.maximum(x, 1e-12)
    i = plsc.bitcast(x, jnp.int32)
    i = jnp.int32(0x5F3759DF) - lax.shift_right_logical(i, 1)
    r = plsc.bitcast(i, jnp.float32)
    for _ in range(2):
        r = r * (1.5 - 0.5 * x * r * r)
    return x * r


def _tree_sum(vals):
    vals = list(vals)
    while len(vals) > 1:
        nxt = [a + b for a, b in zip(vals[::2], vals[1::2])]
        if len(vals) % 2:
            nxt.append(vals[-1])
        vals = nxt
    return vals[0]


_HI = 24
_LO = _C - _HI


def _sc_body(p_hbm, g_hbm, out_hbm,
             ph_a, pl_a, gh_a, gl_a, ph_b, pl_b, gh_b, gl_b, acc_ref,
             sem_a, sem_b):
    cid = lax.axis_index("c")
    sid = lax.axis_index("s")
    w = sid * 2 + cid

    def issue(t, p_hi, p_lo, g_hi, g_lo, sem):
        t = jnp.minimum(t, _NTASK - 1)
        cell = lax.shift_right_logical(t, _SH)
        b0 = lax.bitwise_and(t, _NBP - 1) * _BW
        pltpu.async_copy(p_hbm.at[cell, pl.ds(0, _HI), pl.ds(b0, _BW)],
                         p_hi, sem)
        pltpu.async_copy(p_hbm.at[cell, pl.ds(_HI, _LO), pl.ds(b0, _BW)],
                         p_lo, sem)
        pltpu.async_copy(g_hbm.at[cell, pl.ds(0, _HI), pl.ds(b0, _BW)],
                         g_hi, sem)
        pltpu.async_copy(g_hbm.at[cell, pl.ds(_HI, _LO), pl.ds(b0, _BW)],
                         g_lo, sem)

    def drain(p_hi, p_lo, g_hi, g_lo, sem):
        dummy_hi = p_hbm.at[0, pl.ds(0, _HI), pl.ds(0, _BW)]
        dummy_lo = p_hbm.at[0, pl.ds(_HI, _LO), pl.ds(0, _BW)]
        pltpu.make_async_copy(dummy_hi, p_hi, sem).wait()
        pltpu.make_async_copy(dummy_lo, p_lo, sem).wait()
        pltpu.make_async_copy(dummy_hi, g_hi, sem).wait()
        pltpu.make_async_copy(dummy_lo, g_lo, sem).wait()

    def compute(t, p_hi, p_lo, g_hi, g_lo, acc):
        vf = jnp.where(t < _NTASK, 1.0, 0.0).astype(jnp.float32)
        vh = vf * _LAMB_NOOBJ

        def ld(bufs, c, o):
            hi, lo = bufs
            if c < _HI:
                return hi[c, pl.ds(o, 16)]
            return lo[c - _HI, pl.ds(o, 16)]

        pb = (p_hi, p_lo)
        gb = (g_hi, g_lo)

        def chunk(k, acc):
            o = k * 16
            mg = ld(gb, 4, o)
            m = jnp.where(mg == 1.0, vf, 0.0)
            def sqdiff(c):
                d = ld(pb, c, o) - ld(gb, c, o)
                return d * d

            s_xy = _tree_sum([sqdiff(c) for c in (0, 1, 5, 6)])
            wh = []
            for c in (2, 3, 7, 8):
                pv = ld(pb, c, o)
                gv = ld(gb, c, o)
                wh.append(pv + gv - 2.0 * _sqrt16(pv * gv))
            s_wh = _tree_sum(wh)
            d4 = ld(pb, 4, o) - mg
            d9 = ld(pb, 9, o) - ld(gb, 9, o)
            s_conf = d4 * d4 + d9 * d9
            s_cls = _tree_sum([sqdiff(c) for c in range(10, 30)])
            return acc + (m * (_LAMB_COORD * (s_xy + s_wh) + s_cls
                               + (1.0 - _LAMB_NOOBJ) * s_conf)
                          + vh * s_conf)

        return lax.fori_loop(0, _BW // 16, chunk, acc, unroll=False)

    issue(w, ph_a, pl_a, gh_a, gl_a, sem_a)

    def pair_body(jj, acc):
        t0 = w + _NW * 2 * jj
        issue(t0 + _NW, ph_b, pl_b, gh_b, gl_b, sem_b)
        drain(ph_a, pl_a, gh_a, gl_a, sem_a)
        acc = compute(t0, ph_a, pl_a, gh_a, gl_a, acc)
        issue(t0 + 2 * _NW, ph_a, pl_a, gh_a, gl_a, sem_a)
        drain(ph_b, pl_b, gh_b, gl_b, sem_b)
        acc = compute(t0 + _NW, ph_b, pl_b, gh_b, gl_b, acc)
        return acc

    acc = lax.fori_loop(0, _PAIRS, pair_body,
                        jnp.zeros((16,), jnp.float32), unroll=False)
    drain(ph_a, pl_a, gh_a, gl_a, sem_a)
    acc = compute(w + 2 * _PAIRS * _NW, ph_a, pl_a, gh_a, gl_a, acc)
    acc_ref[...] = acc
    pltpu.sync_copy(acc_ref, out_hbm.at[pl.ds(w * 16, 16)])


@jax.jit
def _sc_loss(pt, gtt):
    mesh = plsc.VectorSubcoreMesh(core_axis_name="c", subcore_axis_name="s")
    run = pl.kernel(
        _sc_body,
        out_type=jax.ShapeDtypeStruct((_NW * 16,), jnp.float32),
        mesh=mesh,
        scratch_types=[
            pltpu.VMEM((_HI, _BW), jnp.float32),
            pltpu.VMEM((_LO, _BW), jnp.float32),
            pltpu.VMEM((_HI, _BW), jnp.float32),
            pltpu.VMEM((_LO, _BW), jnp.float32),
            pltpu.VMEM((_HI, _BW), jnp.float32),
            pltpu.VMEM((_LO, _BW), jnp.float32),
            pltpu.VMEM((_HI, _BW), jnp.float32),
            pltpu.VMEM((_LO, _BW), jnp.float32),
            pltpu.VMEM((16,), jnp.float32),
            pltpu.SemaphoreType.DMA,
            pltpu.SemaphoreType.DMA,
        ],
        compiler_params=pltpu.CompilerParams(
            needs_layout_passes=False, use_tc_tiling_on_sc=True),
    )
    return run(pt, gtt)


def kernel(pred, gt):
    b = pred.shape[0]
    pt = jnp.transpose(pred, (2, 3, 1, 0)).reshape(_S2, _C, _B)
    gtt = jnp.transpose(gt, (2, 3, 1, 0)).reshape(_S2, _C, _B)
    partials = _sc_loss(pt, gtt)
    return jnp.sum(partials) / b


# speedup vs baseline: 1.0593x; 1.0593x over previous
import jax
import jax.numpy as jnp
from jax import lax
from jax.experimental import pallas as pl
from jax.experimental.pallas import tpu as pltpu
from jax.experimental.pallas import tpu_sc as plsc

_LAMB_COORD = 5.0
_LAMB_NOOBJ = 0.5
_B, _C, _S = 2048, 30, 7
_S2 = _S * _S
_BW = 512
_NBP = _B // _BW
_NTASK = _S2 * _NBP
_NW = 32
_SH = 2
_PAIRS = 3


def _sqrt16(x):
    x = jnp.maximum(x, 1e-12)
    i = plsc.bitcast(x, jnp.int32)
    i = jnp.int32(0x5F3759DF) - lax.shift_right_logical(i, 1)
    r = plsc.bitcast(i, jnp.float32)
    for _ in range(2):
        r = r * (1.5 - 0.5 * x * r * r)
    return x * r


def _tree_sum(vals):
    vals = list(vals)
    while len(vals) > 1:
        nxt = [a + b for a, b in zip(vals[::2], vals[1::2])]
        if len(vals) % 2:
            nxt.append(vals[-1])
        vals = nxt
    return vals[0]


_HI = 24
_LO = _C - _HI


def _sc_body(p_hbm, g_hbm, out_hbm,
             ph_a, pl_a, gh_a, gl_a, ph_b, pl_b, gh_b, gl_b, acc_ref,
             sem_a, sem_b):
    cid = lax.axis_index("c")
    sid = lax.axis_index("s")
    w = sid * 2 + cid

    def issue(t, p_hi, p_lo, g_hi, g_lo, sem):
        @pl.when(t < _NTASK)
        def _():
            cell = lax.shift_right_logical(t, _SH)
            b0 = lax.bitwise_and(t, _NBP - 1) * _BW
            pltpu.async_copy(p_hbm.at[cell, pl.ds(0, _HI), pl.ds(b0, _BW)],
                             p_hi, sem)
            pltpu.async_copy(p_hbm.at[cell, pl.ds(_HI, _LO), pl.ds(b0, _BW)],
                             p_lo, sem)
            pltpu.async_copy(g_hbm.at[cell, pl.ds(0, _HI), pl.ds(b0, _BW)],
                             g_hi, sem)
            pltpu.async_copy(g_hbm.at[cell, pl.ds(_HI, _LO), pl.ds(b0, _BW)],
                             g_lo, sem)

    def drain(t, p_hi, p_lo, g_hi, g_lo, sem):
        @pl.when(t < _NTASK)
        def _():
            dummy_hi = p_hbm.at[0, pl.ds(0, _HI), pl.ds(0, _BW)]
            dummy_lo = p_hbm.at[0, pl.ds(_HI, _LO), pl.ds(0, _BW)]
            pltpu.make_async_copy(dummy_hi, p_hi, sem).wait()
            pltpu.make_async_copy(dummy_lo, p_lo, sem).wait()
            pltpu.make_async_copy(dummy_hi, g_hi, sem).wait()
            pltpu.make_async_copy(dummy_lo, g_lo, sem).wait()

    def compute(t, p_hi, p_lo, g_hi, g_lo, acc):
        vf = jnp.float32(1.0)
        vh = vf * _LAMB_NOOBJ

        def ld(bufs, c, o):
            hi, lo = bufs
            if c < _HI:
                return hi[c, pl.ds(o, 16)]
            return lo[c - _HI, pl.ds(o, 16)]

        pb = (p_hi, p_lo)
        gb = (g_hi, g_lo)

        def chunk(k, acc):
            o = k * 16
            mg = ld(gb, 4, o)
            m = jnp.where(mg == 1.0, vf, 0.0)
            def sqdiff(c):
                d = ld(pb, c, o) - ld(gb, c, o)
                return d * d

            s_xy = _tree_sum([sqdiff(c) for c in (0, 1, 5, 6)])
            wh = []
            for c in (2, 3, 7, 8):
                pv = ld(pb, c, o)
                gv = ld(gb, c, o)
                wh.append(pv + gv - 2.0 * _sqrt16(pv * gv))
            s_wh = _tree_sum(wh)
            d4 = ld(pb, 4, o) - mg
            d9 = ld(pb, 9, o) - ld(gb, 9, o)
            s_conf = d4 * d4 + d9 * d9
            s_cls = _tree_sum([sqdiff(c) for c in range(10, 30)])
            return acc + (m * (_LAMB_COORD * (s_xy + s_wh) + s_cls
                               + (1.0 - _LAMB_NOOBJ) * s_conf)
                          + vh * s_conf)

        return lax.fori_loop(0, _BW // 16, chunk, acc, unroll=False)

    def compute_if(t, p_hi, p_lo, g_hi, g_lo, acc):
        return lax.cond(t < _NTASK,
                        lambda a: compute(t, p_hi, p_lo, g_hi, g_lo, a),
                        lambda a: a, acc)

    issue(w, ph_a, pl_a, gh_a, gl_a, sem_a)

    def pair_body(jj, acc):
        t0 = w + _NW * 2 * jj
        issue(t0 + _NW, ph_b, pl_b, gh_b, gl_b, sem_b)
        drain(t0, ph_a, pl_a, gh_a, gl_a, sem_a)
        acc = compute_if(t0, ph_a, pl_a, gh_a, gl_a, acc)
        issue(t0 + 2 * _NW, ph_a, pl_a, gh_a, gl_a, sem_a)
        drain(t0 + _NW, ph_b, pl_b, gh_b, gl_b, sem_b)
        acc = compute_if(t0 + _NW, ph_b, pl_b, gh_b, gl_b, acc)
        return acc

    acc = lax.fori_loop(0, _PAIRS, pair_body,
                        jnp.zeros((16,), jnp.float32), unroll=False)
    t_ep = w + 2 * _PAIRS * _NW
    drain(t_ep, ph_a, pl_a, gh_a, gl_a, sem_a)
    acc = compute_if(t_ep, ph_a, pl_a, gh_a, gl_a, acc)
    acc_ref[...] = acc
    pltpu.sync_copy(acc_ref, out_hbm.at[pl.ds(w * 16, 16)])


@jax.jit
def _sc_loss(pt, gtt):
    mesh = plsc.VectorSubcoreMesh(core_axis_name="c", subcore_axis_name="s")
    run = pl.kernel(
        _sc_body,
        out_type=jax.ShapeDtypeStruct((_NW * 16,), jnp.float32),
        mesh=mesh,
        scratch_types=[
            pltpu.VMEM((_HI, _BW), jnp.float32),
            pltpu.VMEM((_LO, _BW), jnp.float32),
            pltpu.VMEM((_HI, _BW), jnp.float32),
            pltpu.VMEM((_LO, _BW), jnp.float32),
            pltpu.VMEM((_HI, _BW), jnp.float32),
            pltpu.VMEM((_LO, _BW), jnp.float32),
            pltpu.VMEM((_HI, _BW), jnp.float32),
            pltpu.VMEM((_LO, _BW), jnp.float32),
            pltpu.VMEM((16,), jnp.float32),
            pltpu.SemaphoreType.DMA,
            pltpu.SemaphoreType.DMA,
        ],
        compiler_params=pltpu.CompilerParams(
            needs_layout_passes=False, use_tc_tiling_on_sc=True),
    )
    return run(pt, gtt)


def kernel(pred, gt):
    b = pred.shape[0]
    pt = jnp.transpose(pred, (2, 3, 1, 0)).reshape(_S2, _C, _B)
    gtt = jnp.transpose(gt, (2, 3, 1, 0)).reshape(_S2, _C, _B)
    partials = _sc_loss(pt, gtt)
    return jnp.sum(partials) / b
